# baseline (speedup 1.0000x reference)
import functools

import jax
import jax.numpy as jnp
from jax import lax
from jax.experimental import pallas as pl
from jax.experimental.pallas import tpu as pltpu
from jax.experimental.pallas import tpu_sc as plsc

_VOCAB = 50257
_ADDED_OFF = 50257
_VQ_START = 50769
_EMBED = 1024
_VQ_DIM = 256
_VQ_VOCAB = 8192
_N_ADDED = 512
_EXT_ROWS = _N_ADDED + _VQ_VOCAB

_TOKENS = 4 * 8192
_B = 256


def _build_ext(W_add, W_cb, W_proj):
    def body(wadd_ref, wcb_ref, wproj_ref, out_ref):
        i = pl.program_id(0)

        @pl.when(i == 0)
        def _():
            out_ref[...] = wadd_ref[...]

        @pl.when(i > 0)
        def _():
            out_ref[...] = lax.dot_general(
                wcb_ref[...],
                wproj_ref[...],
                (((1,), (1,)), ((), ())),
                preferred_element_type=jnp.float32,
            )

    return pl.pallas_call(
        body,
        grid=(_EXT_ROWS // _N_ADDED,),
        in_specs=[
            pl.BlockSpec((_N_ADDED, _EMBED), lambda i: (0, 0)),
            pl.BlockSpec((_N_ADDED, _VQ_DIM), lambda i: (jnp.maximum(i - 1, 0), 0)),
            pl.BlockSpec((_EMBED, _VQ_DIM), lambda i: (0, 0)),
        ],
        out_specs=pl.BlockSpec((_N_ADDED, _EMBED), lambda i: (i, 0)),
        out_shape=jax.ShapeDtypeStruct((_EXT_ROWS, _EMBED), jnp.float32),
    )(W_add, W_cb, W_proj)


def _tc_gather(x_flat, W_tok, W_ext):
    tok3 = W_tok.reshape(_VOCAB, 8, 128)
    ext3 = W_ext.reshape(_EXT_ROWS, 8, 128)

    def body(ids_ref, tok_hbm, ext_hbm, out_ref, sem):
        def row(j, carry):
            v = ids_ref[j]

            @pl.when(v < _ADDED_OFF)
            def _():
                pltpu.make_async_copy(
                    tok_hbm.at[pl.ds(v, 1)], out_ref.at[pl.ds(j, 1)], sem
                ).start()

            @pl.when(v >= _ADDED_OFF)
            def _():
                pltpu.make_async_copy(
                    ext_hbm.at[pl.ds(v - _ADDED_OFF, 1)],
                    out_ref.at[pl.ds(j, 1)],
                    sem,
                ).start()

            return carry

        lax.fori_loop(0, _B, row, 0)
        pltpu.make_async_copy(tok_hbm.at[pl.ds(0, _B)], out_ref, sem).wait()

    out = pl.pallas_call(
        body,
        grid=(_TOKENS // _B,),
        in_specs=[
            pl.BlockSpec((_B,), lambda i: (i,),
                         memory_space=pltpu.MemorySpace.SMEM),
            pl.BlockSpec(memory_space=pltpu.MemorySpace.HBM),
            pl.BlockSpec(memory_space=pltpu.MemorySpace.HBM),
        ],
        out_specs=pl.BlockSpec((_B, 8, 128), lambda i: (i, 0, 0)),
        out_shape=jax.ShapeDtypeStruct((_TOKENS, 8, 128), jnp.float32),
        scratch_shapes=[pltpu.SemaphoreType.DMA],
    )(x_flat, tok3, ext3)
    return out.reshape(_TOKENS, _EMBED)


def kernel(x, W_tok, W_add, W_cb, W_proj):
    W_ext = _build_ext(W_add, W_cb, W_proj)
    out = _tc_gather(x.reshape(-1), W_tok, W_ext)
    return out.reshape(x.shape + (_EMBED,))
